# Initial kernel scaffold; baseline (speedup 1.0000x reference)
import jax
import jax.numpy as jnp
from jax.experimental import pallas as pl


def kernel(x, edge_index, W1_0, b1_0, g0, be0, rm0, rv0, W2_0, b2_0, W1s, b1s, gs, bes, rms, rvs, W2s, b2s, Wm, bm, Ws, bstd):
    raise NotImplementedError("write your pallas kernel here")


# speedup vs baseline: 10.2850x; 10.2850x over previous
import functools

import jax
import jax.numpy as jnp
from jax import lax
from jax.experimental import pallas as pl
from jax.experimental.pallas import tpu as pltpu
from jax.experimental.pallas import tpu_sc as plsc

N = 10000
H = 16
LAT = 8
BN_EPS = 1e-5

NC = 2
NS = 16
NTILES = NC * NS
CHUNK = 128
ROWS_PER_SUB = 632
NACC = NS * ROWS_PER_SUB


def _make_segsum(nchunk):
    def body(h_hbm, srcm_hbm, dstm_hbm, zeros_hbm, out_hbm,
             srcm_v, dstm_v, rows_v, acc_sh):
        c = lax.axis_index("c")
        s = lax.axis_index("s")
        wid = c * NS + s
        pltpu.sync_copy(srcm_hbm.at[wid], srcm_v)
        pltpu.sync_copy(dstm_hbm.at[wid], dstm_v)
        pltpu.sync_copy(zeros_hbm, acc_sh.at[pl.ds(s * ROWS_PER_SUB, ROWS_PER_SUB)])
        plsc.subcore_barrier()

        def step(j, carry):
            pltpu.sync_copy(h_hbm.at[srcm_v.at[j]], rows_v)
            pltpu.sync_copy(rows_v, acc_sh.at[dstm_v.at[j]], add=True)
            return carry

        lax.fori_loop(0, nchunk, step, 0)
        plsc.subcore_barrier()
        pltpu.sync_copy(acc_sh.at[pl.ds(s * ROWS_PER_SUB, ROWS_PER_SUB)],
                        out_hbm.at[c, pl.ds(s * ROWS_PER_SUB, ROWS_PER_SUB)])

    return pl.kernel(
        body,
        out_type=jax.ShapeDtypeStruct((NC, NACC, H), jnp.float32),
        mesh=plsc.VectorSubcoreMesh(core_axis_name="c", subcore_axis_name="s"),
        compiler_params=pltpu.CompilerParams(use_tc_tiling_on_sc=False),
        scratch_types=[
            pltpu.VMEM((nchunk, CHUNK), jnp.int32),
            pltpu.VMEM((nchunk, CHUNK), jnp.int32),
            pltpu.VMEM((CHUNK, H), jnp.float32),
            pltpu.VMEM_SHARED((NACC, H), jnp.float32),
        ],
    )


ROW_BLK = 1000


def _proj_body(x_ref, w_ref, o_ref):
    o_ref[...] = jnp.dot(x_ref[...], w_ref[...],
                         preferred_element_type=jnp.float32)


def _proj(x, w):
    n, d = x.shape
    return pl.pallas_call(
        _proj_body,
        grid=(n // ROW_BLK,),
        in_specs=[
            pl.BlockSpec((ROW_BLK, d), lambda i: (i, 0)),
            pl.BlockSpec((d, H), lambda i: (0, 0)),
        ],
        out_specs=pl.BlockSpec((ROW_BLK, H), lambda i: (i, 0)),
        out_shape=jax.ShapeDtypeStruct((n, H), jnp.float32),
    )(x, w)


def _epi_body(h_ref, p0_ref, p1_ref, w1_ref, b1_ref, a_ref, bb_ref,
              w2_ref, b2_ref, o_ref, *, relu_out):
    t = h_ref[...] + p0_ref[...] + p1_ref[...]
    u = jnp.dot(t, w1_ref[...], preferred_element_type=jnp.float32) + b1_ref[...]
    u = u * a_ref[...] + bb_ref[...]
    u = jnp.maximum(u, 0.0)
    v = jnp.dot(u, w2_ref[...], preferred_element_type=jnp.float32) + b2_ref[...]
    if relu_out:
        v = jnp.maximum(v, 0.0)
    o_ref[...] = v


def _epilogue(h, p0, p1, w1, b1, a, bb, w2, b2, relu_out):
    mat = lambda: pl.BlockSpec((H, H), lambda i: (0, 0))
    vec = lambda: pl.BlockSpec((1, H), lambda i: (0, 0))
    big = lambda: pl.BlockSpec((ROW_BLK, H), lambda i: (i, 0))
    return pl.pallas_call(
        functools.partial(_epi_body, relu_out=relu_out),
        grid=(N // ROW_BLK,),
        in_specs=[big(), big(), big(), mat(), vec(), vec(), vec(), mat(), vec()],
        out_specs=big(),
        out_shape=jax.ShapeDtypeStruct((N, H), jnp.float32),
    )(h, p0, p1, w1, b1.reshape(1, H), a.reshape(1, H), bb.reshape(1, H),
      w2, b2.reshape(1, H))


def _final_body(h_ref, p0_ref, p1_ref, w1_ref, b1_ref, a_ref, bb_ref,
                w2_ref, b2_ref, wh_ref, bh_ref, o_ref):
    t = h_ref[...] + p0_ref[...] + p1_ref[...]
    u = jnp.dot(t, w1_ref[...], preferred_element_type=jnp.float32) + b1_ref[...]
    u = u * a_ref[...] + bb_ref[...]
    u = jnp.maximum(u, 0.0)
    v = jnp.dot(u, w2_ref[...], preferred_element_type=jnp.float32) + b2_ref[...]
    z = jnp.dot(v, wh_ref[...], preferred_element_type=jnp.float32) + bh_ref[...]
    col = lax.broadcasted_iota(jnp.int32, z.shape, 1)
    o_ref[...] = jnp.where(col < LAT, z, jax.nn.softplus(z))


def _final(h, p0, p1, w1, b1, a, bb, w2, b2, wh, bh):
    mat = lambda: pl.BlockSpec((H, H), lambda i: (0, 0))
    vec = lambda: pl.BlockSpec((1, H), lambda i: (0, 0))
    big = lambda: pl.BlockSpec((ROW_BLK, H), lambda i: (i, 0))
    return pl.pallas_call(
        _final_body,
        grid=(N // ROW_BLK,),
        in_specs=[big(), big(), big(), mat(), vec(), vec(), vec(), mat(),
                  vec(), mat(), vec()],
        out_specs=big(),
        out_shape=jax.ShapeDtypeStruct((N, H), jnp.float32),
    )(h, p0, p1, w1, b1.reshape(1, H), a.reshape(1, H), bb.reshape(1, H),
      w2, b2.reshape(1, H), wh, bh.reshape(1, H))


def kernel(x, edge_index, W1_0, b1_0, g0, be0, rm0, rv0, W2_0, b2_0,
           W1s, b1s, gs, bes, rms, rvs, W2s, b2s, Wm, bm, Ws, bstd):
    src = edge_index[0]
    dst = edge_index[1]
    e = src.shape[0]
    nchunk = -(-e // (NTILES * CHUNK))
    epad = NTILES * nchunk * CHUNK
    src_p = jnp.concatenate([src, jnp.zeros((epad - e,), jnp.int32)])
    dst_p = jnp.concatenate([dst, jnp.full((epad - e,), N, jnp.int32)])
    srcm = src_p.reshape(NTILES, nchunk, CHUNK)
    dstm = dst_p.reshape(NTILES, nchunk, CHUNK)
    zeros = jnp.zeros((ROWS_PER_SUB, H), jnp.float32)

    segsum = _make_segsum(nchunk)

    p = _proj(x, W1_0)
    parts = segsum(p, srcm, dstm, zeros)
    a0 = g0 * lax.rsqrt(rv0 + BN_EPS)
    bb0 = be0 - rm0 * a0
    h = _epilogue(p, parts[0, :N], parts[1, :N], jnp.eye(H, dtype=jnp.float32),
                  b1_0, a0, bb0, W2_0, b2_0, relu_out=True)

    for i in range(9):
        parts = segsum(h, srcm, dstm, zeros)
        ai = gs[i] * lax.rsqrt(rvs[i] + BN_EPS)
        bbi = bes[i] - rms[i] * ai
        if i < 8:
            h = _epilogue(h, parts[0, :N], parts[1, :N], W1s[i], b1s[i],
                          ai, bbi, W2s[i], b2s[i], relu_out=True)
        else:
            wh = jnp.concatenate([Wm, Ws], axis=1)
            bh = jnp.concatenate([bm, bstd])
            out = _final(h, parts[0, :N], parts[1, :N], W1s[i], b1s[i],
                         ai, bbi, W2s[i], b2s[i], wh, bh)
    return (out[:, :LAT], out[:, LAT:])
